# baseline (device time: 24577 ns/iter reference)
import jax
import jax.numpy as jnp
from jax import lax
from jax.experimental import pallas as pl
from jax.experimental.pallas import tpu as pltpu

N_CHUNKS = 16
N_DY = 4
N_MM = 2


def kernel(x, dy):
    k_dim, d = x.shape
    _, f = dy.shape
    m_out = d // 2
    half = m_out // 2
    cw = f // N_CHUNKS
    dw = f // N_DY
    mw = f // N_MM

    dn = (((0,), (0,)), ((), ()))

    def body(x_ref, dy_ref, out_ref, xs_buf, xm_buf, dy_buf,
             send_buf, l_buf, recv_y,
             copy_sems, send_sems1, recv_sems1, send_sems2, recv_sems2):
        my_x = lax.axis_index("x")
        my_y = lax.axis_index("y")

        barrier = pltpu.get_barrier_semaphore()
        pl.semaphore_signal(barrier, inc=1, device_id=(my_x, 1 - my_y),
                            device_id_type=pl.DeviceIdType.MESH)
        pl.semaphore_signal(barrier, inc=1, device_id=(1 - my_x, my_y),
                            device_id_type=pl.DeviceIdType.MESH)

        c_send = (1 - my_y) * m_out + my_x * half
        c_loc = my_y * m_out + my_x * half
        cp_xs = pltpu.make_async_copy(
            x_ref.at[:, pl.ds(c_send, half)], xs_buf, copy_sems.at[0])
        cp_xm = pltpu.make_async_copy(
            x_ref.at[:, pl.ds(c_loc, half)], xm_buf, copy_sems.at[1])
        cp_xs.start()
        cp_xm.start()
        cp_dy = []
        for dd in range(N_DY):
            sl = pl.ds(dd * dw, dw)
            cp = pltpu.make_async_copy(
                dy_ref.at[:, sl], dy_buf.at[:, sl], copy_sems.at[2 + dd])
            cp.start()
            cp_dy.append(cp)

        cp_xs.wait()
        xs = xs_buf[...]

        rdma1 = []
        for mm in range(N_MM):
            for dd in range(mm * N_DY // N_MM, (mm + 1) * N_DY // N_MM):
                cp_dy[dd].wait()
            sl = pl.ds(mm * mw, mw)
            send_buf[:, sl] = lax.dot_general(
                xs, dy_buf[:, sl], dn, preferred_element_type=jnp.float32)
            if mm == 0:
                pl.semaphore_wait(barrier, 2)
            for kk in range(mm * N_CHUNKS // N_MM,
                            (mm + 1) * N_CHUNKS // N_MM):
                slk = pl.ds(kk * cw, cw)
                r = pltpu.make_async_remote_copy(
                    src_ref=send_buf.at[:, slk], dst_ref=recv_y.at[:, slk],
                    send_sem=send_sems1.at[kk], recv_sem=recv_sems1.at[kk],
                    device_id=(my_x, 1 - my_y),
                    device_id_type=pl.DeviceIdType.MESH)
                r.start()
                rdma1.append(r)

        cp_xm.wait()
        l_buf[...] = lax.dot_general(
            xm_buf[...], dy_buf[...], dn, preferred_element_type=jnp.float32)

        r0 = my_x * half

        rdma2 = []
        for kk in range(N_CHUNKS):
            sl = pl.ds(kk * cw, cw)
            rdma1[kk].wait()
            out_ref[pl.ds(r0, half), sl] = l_buf[:, sl] + recv_y[:, sl]
            r = pltpu.make_async_remote_copy(
                src_ref=out_ref.at[pl.ds(r0, half), sl],
                dst_ref=out_ref.at[pl.ds(r0, half), sl],
                send_sem=send_sems2.at[kk], recv_sem=recv_sems2.at[kk],
                device_id=(1 - my_x, my_y),
                device_id_type=pl.DeviceIdType.MESH)
            r.start()
            rdma2.append(r)

        for kk in range(N_CHUNKS):
            rdma2[kk].wait()

    return pl.pallas_call(
        body,
        out_shape=jax.ShapeDtypeStruct((m_out, f), jnp.float32),
        in_specs=[pl.BlockSpec(memory_space=pl.ANY),
                  pl.BlockSpec(memory_space=pl.ANY)],
        out_specs=pl.BlockSpec(memory_space=pltpu.VMEM),
        scratch_shapes=[
            pltpu.VMEM((k_dim, half), jnp.float32),
            pltpu.VMEM((k_dim, half), jnp.float32),
            pltpu.VMEM((k_dim, f), jnp.float32),
            pltpu.VMEM((half, f), jnp.float32),
            pltpu.VMEM((half, f), jnp.float32),
            pltpu.VMEM((half, f), jnp.float32),
            pltpu.SemaphoreType.DMA((2 + N_DY,)),
            pltpu.SemaphoreType.DMA((N_CHUNKS,)),
            pltpu.SemaphoreType.DMA((N_CHUNKS,)),
            pltpu.SemaphoreType.DMA((N_CHUNKS,)),
            pltpu.SemaphoreType.DMA((N_CHUNKS,)),
        ],
        compiler_params=pltpu.CompilerParams(collective_id=0),
    )(x, dy)


# device time: 23731 ns/iter; 1.0356x vs baseline; 1.0356x over previous
import jax
import jax.numpy as jnp
from jax import lax
from jax.experimental import pallas as pl
from jax.experimental.pallas import tpu as pltpu

N_CHUNKS = 16


def kernel(x, dy):
    k_dim, d = x.shape
    _, f = dy.shape
    m_out = d // 2
    half = m_out // 2
    cw = f // N_CHUNKS

    dn = (((0,), (0,)), ((), ()))

    def body(x_ref, dy_ref, out_ref, send_buf, l_buf, recv_y,
             send_sems1, recv_sems1, send_sems2, recv_sems2):
        my_x = lax.axis_index("x")
        my_y = lax.axis_index("y")

        barrier = pltpu.get_barrier_semaphore()
        pl.semaphore_signal(barrier, inc=1, device_id=(my_x, 1 - my_y),
                            device_id_type=pl.DeviceIdType.MESH)
        pl.semaphore_signal(barrier, inc=1, device_id=(1 - my_x, my_y),
                            device_id_type=pl.DeviceIdType.MESH)

        c_send = (1 - my_y) * m_out + my_x * half
        xs = x_ref[:, pl.ds(c_send, half)]
        send_buf[...] = lax.dot_general(
            xs, dy_ref[...], dn, preferred_element_type=jnp.float32)
        pl.semaphore_wait(barrier, 2)

        rdma1 = []
        for kk in range(N_CHUNKS):
            sl = pl.ds(kk * cw, cw)
            r = pltpu.make_async_remote_copy(
                src_ref=send_buf.at[:, sl], dst_ref=recv_y.at[:, sl],
                send_sem=send_sems1.at[kk], recv_sem=recv_sems1.at[kk],
                device_id=(my_x, 1 - my_y),
                device_id_type=pl.DeviceIdType.MESH)
            r.start()
            rdma1.append(r)

        c_loc = my_y * m_out + my_x * half
        xm = x_ref[:, pl.ds(c_loc, half)]
        l_buf[...] = lax.dot_general(
            xm, dy_ref[...], dn, preferred_element_type=jnp.float32)

        r0 = my_x * half

        rdma2 = []
        for kk in range(N_CHUNKS):
            sl = pl.ds(kk * cw, cw)
            rdma1[kk].wait()
            out_ref[pl.ds(r0, half), sl] = l_buf[:, sl] + recv_y[:, sl]
            r = pltpu.make_async_remote_copy(
                src_ref=out_ref.at[pl.ds(r0, half), sl],
                dst_ref=out_ref.at[pl.ds(r0, half), sl],
                send_sem=send_sems2.at[kk], recv_sem=recv_sems2.at[kk],
                device_id=(1 - my_x, my_y),
                device_id_type=pl.DeviceIdType.MESH)
            r.start()
            rdma2.append(r)

        for kk in range(N_CHUNKS):
            rdma2[kk].wait()

    return pl.pallas_call(
        body,
        out_shape=jax.ShapeDtypeStruct((m_out, f), jnp.float32),
        in_specs=[pl.BlockSpec(memory_space=pltpu.VMEM),
                  pl.BlockSpec(memory_space=pltpu.VMEM)],
        out_specs=pl.BlockSpec(memory_space=pltpu.VMEM),
        scratch_shapes=[
            pltpu.VMEM((half, f), jnp.float32),
            pltpu.VMEM((half, f), jnp.float32),
            pltpu.VMEM((half, f), jnp.float32),
            pltpu.SemaphoreType.DMA((N_CHUNKS,)),
            pltpu.SemaphoreType.DMA((N_CHUNKS,)),
            pltpu.SemaphoreType.DMA((N_CHUNKS,)),
            pltpu.SemaphoreType.DMA((N_CHUNKS,)),
        ],
        compiler_params=pltpu.CompilerParams(collective_id=0),
    )(x, dy)
